# Initial kernel scaffold; baseline (speedup 1.0000x reference)
import jax
import jax.numpy as jnp
from jax.experimental import pallas as pl


def kernel(x, edge_index, edge_weight, filter_param, chebynodes_vals):
    raise NotImplementedError("write your pallas kernel here")


# speedup vs baseline: 3.9292x; 3.9292x over previous
import functools

import jax
import jax.numpy as jnp
from jax import lax
from jax.experimental import pallas as pl
from jax.experimental.pallas import tpu as pltpu
from jax.experimental.pallas import tpu_sc as plsc

K = 8
N = 10000
E = 320000
D = 128

NC = 2
NS = 16
L = 16
NW = NC * NS
CH = 128
EPT = -(-E // NW)
G = -(-EPT // CH)
E_PAD = NW * G * CH
N_ACC = 10240
DUMMY = N_ACC - 8
RPT = N_ACC // NS

_mesh = plsc.VectorSubcoreMesh(core_axis_name="c", subcore_axis_name="s")

_GDN = lax.GatherDimensionNumbers(
    offset_dims=(), collapsed_slice_dims=(0,), start_index_map=(0,))


def _splat(vec, lane):
    idx = jnp.full((L, 1), lane, jnp.int32)
    return lax.gather(vec, idx, _GDN, (1,),
                      mode=lax.GatherScatterMode.PROMISE_IN_BOUNDS)


@functools.partial(
    pl.kernel,
    out_type=jax.ShapeDtypeStruct((NC, N_ACC, D), jnp.float32),
    mesh=_mesh,
    scratch_types=[
        pltpu.VMEM((G, CH), jnp.int32),
        pltpu.VMEM((G, CH), jnp.int32),
        pltpu.VMEM((G * CH,), jnp.float32),
        pltpu.VMEM((CH, D), jnp.float32),
        pltpu.VMEM_SHARED((N_ACC, D), jnp.float32),
        pltpu.SemaphoreType.DMA,
    ],
)
def _spmv(src, col3, row3, w3, zrows, part, colv, rowv, wv, gbuf, acc, sem):
    c = lax.axis_index("c")
    s = lax.axis_index("s")
    wid = c * NS + s

    pltpu.sync_copy(col3.at[wid], colv)
    pltpu.sync_copy(row3.at[wid], rowv)
    pltpu.sync_copy(w3.at[wid], wv)
    pltpu.sync_copy(zrows, acc.at[pl.ds(s * RPT, RPT)])

    def _neg(r, carry):
        sl = pl.ds(r * L, L)
        wv[sl] = -wv[sl]
        return carry

    lax.fori_loop(0, G * CH // L, _neg, 0)
    plsc.subcore_barrier()

    def _chunk(g, carry):
        pltpu.async_copy(src.at[colv.at[g]], gbuf, sem).wait()

        def _grp(h, carry2):
            wvec = wv[pl.ds(g * CH + h * L, L)]
            for jj in range(L):
                wspl = _splat(wvec, jj)
                j = h * L + jj
                for q in range(D // L):
                    sl = pl.ds(q * L, L)
                    gbuf[j, sl] = gbuf[j, sl] * wspl
            return carry2

        lax.fori_loop(0, CH // L, _grp, 0)
        pltpu.sync_copy(gbuf, acc.at[rowv.at[g]], add=True)
        return carry

    lax.fori_loop(0, G, _chunk, 0)
    plsc.subcore_barrier()

    pltpu.sync_copy(acc.at[pl.ds(s * RPT, RPT)],
                    part.at[c, pl.ds(s * RPT, RPT)])


_RB = 1280


def _combine_first(part_ref, prev_ref, y_ref, fp_ref, t_out, y_out):
    u = part_ref[0] + part_ref[1]
    t_out[...] = u
    y_out[...] = fp_ref[0, 0] * prev_ref[...] + fp_ref[1, 0] * u


def _combine_make(i):
    def body(part_ref, prev_ref, y_ref, fp_ref, t_out, y_out):
        t = 2.0 * (part_ref[0] + part_ref[1]) - prev_ref[...]
        t_out[...] = t
        y_out[...] = y_ref[...] + fp_ref[i, 0] * t
    return body


def _combine(part, prev, y, fp, i):
    body = _combine_first if i == 1 else _combine_make(i)
    return pl.pallas_call(
        body,
        grid=(N_ACC // _RB,),
        in_specs=[
            pl.BlockSpec((NC, _RB, D), lambda r: (0, r, 0)),
            pl.BlockSpec((_RB, D), lambda r: (r, 0)),
            pl.BlockSpec((_RB, D), lambda r: (r, 0)),
            pl.BlockSpec(memory_space=pltpu.SMEM),
        ],
        out_specs=[
            pl.BlockSpec((_RB, D), lambda r: (r, 0)),
            pl.BlockSpec((_RB, D), lambda r: (r, 0)),
        ],
        out_shape=[
            jax.ShapeDtypeStruct((N_ACC, D), jnp.float32),
            jax.ShapeDtypeStruct((N_ACC, D), jnp.float32),
        ],
    )(part, prev, y, fp)


def kernel(x, edge_index, edge_weight, filter_param, chebynodes_vals):
    x_pad = jnp.zeros((N_ACC, D), jnp.float32).at[:N].set(x)
    pad = E_PAD - E
    colp = jnp.concatenate(
        [edge_index[1], jnp.zeros((pad,), jnp.int32)]).reshape(NW, G, CH)
    rowp = jnp.concatenate(
        [edge_index[0], jnp.full((pad,), DUMMY, jnp.int32)]).reshape(NW, G, CH)
    wp = jnp.concatenate(
        [edge_weight, jnp.zeros((pad,), jnp.float32)]).reshape(NW, G * CH)
    zrows = jnp.zeros((RPT, D), jnp.float32)

    fp = jax.nn.relu(filter_param)
    fp = chebynodes_vals @ fp
    fp = 2.0 * fp / (K + 1)
    fp = fp.at[0].set(fp[0] / 2.0)

    prev = x_pad
    cur = x_pad
    y = x_pad
    for i in range(1, K + 1):
        part = _spmv(cur, colp, rowp, wp, zrows)
        t, y = _combine(part, prev, y, fp, i)
        prev, cur = cur, t
    return y[:N]
